# baseline (speedup 1.0000x reference)
import jax
import jax.numpy as jnp
from jax.experimental import pallas as pl

_NODES = 50
_DO = 12
_DA = 4
_OBS_W = _NODES * _DO
_ACT_W = _NODES * _DA
_GRID = 4


def _group_mask(total, d):
    r = jax.lax.broadcasted_iota(jnp.int32, (total, d), 0)
    c = jax.lax.broadcasted_iota(jnp.int32, (total, d), 1)
    return (r % d == c).astype(jnp.float32)


def _body(obs_ref, act_ref,
          W1_1_ref, b1_1_ref, W2_1_ref, b2_1_ref, W3_1_ref, b3_1_ref,
          W1_2_ref, b1_2_ref, W2_2_ref, b2_2_ref, W3_2_ref, b3_2_ref,
          q1_ref, q2_ref):
    blk = obs_ref.shape[0]
    inv = jnp.float32(1.0 / _NODES)
    po = _group_mask(_OBS_W, _DO)
    pa = _group_mask(_ACT_W, _DA)
    mo = jnp.dot(obs_ref[:], po, preferred_element_type=jnp.float32)
    ma = jnp.dot(act_ref[:], pa, preferred_element_type=jnp.float32)
    xm = jnp.concatenate([mo, ma], axis=-1) * inv

    def head(W1, b1, W2, b2, W3, b3):
        h = jnp.dot(xm, W1[:], preferred_element_type=jnp.float32)
        h = jnp.maximum(h + b1[:], 0.0)
        h = jnp.maximum(jnp.dot(h, W2[:], preferred_element_type=jnp.float32) + b2[:], 0.0)
        q = jnp.dot(h, W3[:], preferred_element_type=jnp.float32) + b3[:]
        return jnp.broadcast_to(q, (blk, _NODES))

    q1_ref[:] = head(W1_1_ref, b1_1_ref, W2_1_ref, b2_1_ref, W3_1_ref, b3_1_ref)
    q2_ref[:] = head(W1_2_ref, b1_2_ref, W2_2_ref, b2_2_ref, W3_2_ref, b3_2_ref)


def _const_spec(shape):
    return pl.BlockSpec(shape, lambda i: (0, 0))


def kernel(obs, action, W1_q1, b1_q1, W2_q1, b2_q1, W3_q1, b3_q1,
           W1_q2, b1_q2, W2_q2, b2_q2, W3_q2, b3_q2):
    bs = obs.shape[0]
    hid = W1_q1.shape[1]
    blk = bs // _GRID
    out_shape = (jax.ShapeDtypeStruct((bs, _NODES), jnp.float32),
                 jax.ShapeDtypeStruct((bs, _NODES), jnp.float32))
    row_spec = lambda w: pl.BlockSpec((blk, w), lambda i: (i, 0))
    wspecs = [
        _const_spec((hid // 4, hid)), _const_spec((1, hid)),
        _const_spec((hid, hid)), _const_spec((1, hid)),
        _const_spec((hid, 1)), _const_spec((1, 1)),
    ] * 2
    q1, q2 = pl.pallas_call(
        _body,
        grid=(_GRID,),
        in_specs=[row_spec(_OBS_W), row_spec(_ACT_W)] + wspecs,
        out_specs=(row_spec(_NODES), row_spec(_NODES)),
        out_shape=out_shape,
    )(
        obs, action,
        W1_q1, b1_q1.reshape(1, hid), W2_q1, b2_q1.reshape(1, hid),
        W3_q1, b3_q1.reshape(1, 1),
        W1_q2, b1_q2.reshape(1, hid), W2_q2, b2_q2.reshape(1, hid),
        W3_q2, b3_q2.reshape(1, 1),
    )
    return (q1, q2)


# speedup vs baseline: 1.2006x; 1.2006x over previous
import jax
import jax.numpy as jnp
from jax.experimental import pallas as pl

_NODES = 50
_DO = 12
_DA = 4


def _group_mask(total, d):
    r = jax.lax.broadcasted_iota(jnp.int32, (total, d), 0)
    c = jax.lax.broadcasted_iota(jnp.int32, (total, d), 1)
    return (r % d == c).astype(jnp.float32)


def _body(obs_ref, act_ref,
          W1_1_ref, b1_1_ref, W2_1_ref, b2_1_ref, W3_1_ref, b3_1_ref,
          W1_2_ref, b1_2_ref, W2_2_ref, b2_2_ref, W3_2_ref, b3_2_ref,
          q1_ref, q2_ref):
    bs = obs_ref.shape[0]
    inv = jnp.float32(1.0 / _NODES)
    po = _group_mask(_NODES * _DO, _DO)
    pa = _group_mask(_NODES * _DA, _DA)
    mo = jnp.dot(obs_ref[:], po, preferred_element_type=jnp.float32)
    ma = jnp.dot(act_ref[:], pa, preferred_element_type=jnp.float32)
    xm = jnp.concatenate([mo, ma], axis=-1) * inv

    def head(W1, b1, W2, b2, W3, b3):
        h = jnp.dot(xm, W1[:], preferred_element_type=jnp.float32)
        h = jnp.maximum(h + b1[:], 0.0)
        h = jnp.maximum(jnp.dot(h, W2[:], preferred_element_type=jnp.float32) + b2[:], 0.0)
        q = jnp.dot(h, W3[:], preferred_element_type=jnp.float32) + b3[:]
        return jnp.broadcast_to(q, (bs, _NODES))

    q1_ref[:] = head(W1_1_ref, b1_1_ref, W2_1_ref, b2_1_ref, W3_1_ref, b3_1_ref)
    q2_ref[:] = head(W1_2_ref, b1_2_ref, W2_2_ref, b2_2_ref, W3_2_ref, b3_2_ref)


def kernel(obs, action, W1_q1, b1_q1, W2_q1, b2_q1, W3_q1, b3_q1,
           W1_q2, b1_q2, W2_q2, b2_q2, W3_q2, b3_q2):
    bs = obs.shape[0]
    hid = W1_q1.shape[1]
    out_shape = (jax.ShapeDtypeStruct((bs, _NODES), jnp.float32),
                 jax.ShapeDtypeStruct((bs, _NODES), jnp.float32))
    q1, q2 = pl.pallas_call(_body, out_shape=out_shape)(
        obs, action,
        W1_q1, b1_q1.reshape(1, hid), W2_q1, b2_q1.reshape(1, hid),
        W3_q1, b3_q1.reshape(1, 1),
        W1_q2, b1_q2.reshape(1, hid), W2_q2, b2_q2.reshape(1, hid),
        W3_q2, b3_q2.reshape(1, 1),
    )
    return (q1, q2)


# speedup vs baseline: 1.2032x; 1.0021x over previous
import jax
import jax.numpy as jnp
from jax.experimental import pallas as pl
from jax.experimental.pallas import tpu as pltpu

_NODES = 50
_DO = 12
_DA = 4


def _group_mask(total, d):
    r = jax.lax.broadcasted_iota(jnp.int32, (total, d), 0)
    c = jax.lax.broadcasted_iota(jnp.int32, (total, d), 1)
    return (r % d == c).astype(jnp.float32)


def _body(obs_ref, act_ref,
          W1_1_ref, b1_1_ref, W2_1_ref, b2_1_ref, W3_1_ref, b3_1_ref,
          W1_2_ref, b1_2_ref, W2_2_ref, b2_2_ref, W3_2_ref, b3_2_ref,
          q1_ref, q2_ref):
    bs = obs_ref.shape[0]
    inv = jnp.float32(1.0 / _NODES)
    po = _group_mask(_NODES * _DO, _DO)
    pa = _group_mask(_NODES * _DA, _DA)
    mo = jnp.dot(obs_ref[:], po, preferred_element_type=jnp.float32)
    ma = jnp.dot(act_ref[:], pa, preferred_element_type=jnp.float32)
    xm = jnp.concatenate([mo, ma], axis=-1) * inv

    def head(W1, b1, W2, b2, W3, b3):
        h = jnp.dot(xm, W1[:], preferred_element_type=jnp.float32)
        h = jnp.maximum(h + b1[:], 0.0)
        h = jnp.maximum(jnp.dot(h, W2[:], preferred_element_type=jnp.float32) + b2[:], 0.0)
        q = jnp.dot(h, W3[:], preferred_element_type=jnp.float32) + b3[0, 0]
        return jnp.broadcast_to(q, (bs, _NODES))

    q1_ref[:] = head(W1_1_ref, b1_1_ref, W2_1_ref, b2_1_ref, W3_1_ref, b3_1_ref)
    q2_ref[:] = head(W1_2_ref, b1_2_ref, W2_2_ref, b2_2_ref, W3_2_ref, b3_2_ref)


def kernel(obs, action, W1_q1, b1_q1, W2_q1, b2_q1, W3_q1, b3_q1,
           W1_q2, b1_q2, W2_q2, b2_q2, W3_q2, b3_q2):
    bs = obs.shape[0]
    hid = W1_q1.shape[1]
    out_shape = (jax.ShapeDtypeStruct((bs, _NODES), jnp.float32),
                 jax.ShapeDtypeStruct((bs, _NODES), jnp.float32))
    vmem = pl.BlockSpec(memory_space=pltpu.MemorySpace.VMEM)
    smem = pl.BlockSpec(memory_space=pltpu.SMEM)
    in_specs = [vmem] * 7 + [smem] + [vmem] * 5 + [smem]
    q1, q2 = pl.pallas_call(_body, out_shape=out_shape, in_specs=in_specs)(
        obs, action,
        W1_q1, b1_q1.reshape(1, hid), W2_q1, b2_q1.reshape(1, hid),
        W3_q1, b3_q1.reshape(1, 1),
        W1_q2, b1_q2.reshape(1, hid), W2_q2, b2_q2.reshape(1, hid),
        W3_q2, b3_q2.reshape(1, 1),
    )
    return (q1, q2)
